# baseline (speedup 1.0000x reference)
import functools

import jax
import jax.numpy as jnp
from jax import lax
from jax.experimental import pallas as pl
from jax.experimental.pallas import tpu as pltpu
from jax.experimental.pallas import tpu_sc as plsc

_POOL = 1000000
_DIM = 64
_BATCH = 16384

_NW = 32
_RANGE = 31232
_W = 512
_NCH = _RANGE // _W
_T0 = _NW * _RANGE
_T1 = _T0 + _W
_DUMP = _BATCH
_SLOTS = 64

_mesh = plsc.VectorSubcoreMesh(core_axis_name="c", subcore_axis_name="s")


@functools.partial(
    pl.kernel,
    mesh=_mesh,
    out_type=jax.ShapeDtypeStruct((_BATCH + 128, 128), jnp.float32),
    scratch_types=[
        pltpu.VMEM((_BATCH + 16,), jnp.int32),
        pltpu.VMEM((_BATCH + 16,), jnp.int32),
        pltpu.VMEM((_DIM, _W), jnp.float32),
        pltpu.VMEM((_DIM, _W), jnp.float32),
        pltpu.VMEM((_SLOTS, 128), jnp.float32),
        pltpu.VMEM((_SLOTS, 128), jnp.float32),
        pltpu.VMEM((1, _SLOTS), jnp.int32),
        pltpu.VMEM((1, _SLOTS), jnp.int32),
        pltpu.VMEM((_DIM, 64), jnp.float32),
        pltpu.SemaphoreType.DMA,
        pltpu.SemaphoreType.DMA,
        pltpu.SemaphoreType.DMA,
        pltpu.SemaphoreType.DMA,
    ],
    compiler_params=pltpu.CompilerParams(
        needs_layout_passes=False, disable_bounds_checks=True),
)
def _lookup(ids_hbm, poolt_hbm, tail_hbm, out_hbm, ids_v, hits_v, buf0, buf1,
            stageA, stageB, posrA, posrB, tailbuf, sem0, sem1, semwA, semwB):
    wid = lax.axis_index("s") * 2 + lax.axis_index("c")
    lo = wid * _RANGE
    hi = lo + _RANGE
    lane = lax.iota(jnp.int32, 16)

    def chunk_dma(g, buf, sem):
        for a in range(8):
            pltpu.async_copy(
                poolt_hbm.at[pl.ds(8 * a, 8), pl.ds(lo + g * _W, _W)],
                buf.at[pl.ds(8 * a, 8), :], sem)

    def drain(buf, sem):
        pltpu.make_async_copy(poolt_hbm.at[:, pl.ds(0, _W)], buf, sem).wait()

    def fire_flush(stage, posr, semw):
        pltpu.async_copy(stage, out_hbm.at[posr.at[0]], semw)

    def drain_flush(stage, semw):
        pltpu.make_async_copy(
            out_hbm.at[pl.ds(0, _SLOTS), :], stage, semw).wait()

    for j in range(_SLOTS // 16):
        posrA[0, pl.ds(16 * j, 16)] = lane + (_DUMP + 16 * j)
        posrB[0, pl.ds(16 * j, 16)] = lane + (_DUMP + 16 * j)
    fire_flush(stageA, posrA, semwA)
    fire_flush(stageB, posrB, semwB)

    chunk_dma(0, buf0, sem0)
    pltpu.sync_copy(ids_hbm, ids_v.at[pl.ds(0, _BATCH)])

    is0 = wid == 0

    def scan_body(i, off):
        v = ids_v[pl.ds(16 * i, 16)]
        in_tail = (v >= _T0) & is0
        m = ((v >= lo) & (v < hi)) | in_tail
        idrel = jnp.where(v >= _T0, _RANGE + (v - _T0), v - lo)
        packed = (idrel << 15) | (lane + 16 * i)
        plsc.store_compressed(hits_v.at[pl.ds(off, 16)], packed, mask=m)
        return off + plsc.all_reduce_population_count(m)[0]

    H = lax.fori_loop(0, _BATCH // 16, scan_body, 0)
    hits_v[pl.ds(H, 16)] = jnp.zeros((16,), jnp.int32) + (1 << 30)
    nvec = (H + 15) // 16

    def work(buf, win_lo, width, stage, posr, semw):
        def filt_body(k, wn):
            hv = hits_v[pl.ds(16 * k, 16)]
            idrel = lax.shift_right_logical(hv, 15)
            m2 = (idrel >= win_lo) & (idrel < win_lo + width)
            plsc.store_compressed(ids_v.at[pl.ds(wn, 16)], hv, mask=m2)
            return wn + plsc.all_reduce_population_count(m2)[0]

        wn = lax.fori_loop(0, nvec, filt_body, 0)
        pad = (win_lo << 15) | _DUMP
        ids_v[pl.ds(wn, 16)] = jnp.zeros((16,), jnp.int32) + pad

        def group_body(t, slotbase):
            hv = ids_v[pl.ds(16 * t, 16)]
            cvec = lax.shift_right_logical(hv, 15) - win_lo
            posv = hv & 32767
            rowvec = slotbase + lane

            def d_body(dd, c):
                for j in range(4):
                    dvec = jnp.zeros((16,), jnp.int32) + (dd * 4 + j)
                    vals = plsc.load_gather(buf, [dvec, cvec])
                    plsc.store_scatter(stage, [rowvec, dvec], vals)
                return c

            lax.fori_loop(0, _DIM // 4, d_body, 0)
            posr[0, pl.ds(slotbase, 16)] = posv
            return slotbase + 16

        ngroups = (wn + 15) // 16
        nfast = jnp.minimum(ngroups, _SLOTS // 16)
        drain_flush(stage, semw)
        lax.fori_loop(0, nfast, group_body, 0)
        fire_flush(stage, posr, semw)

        nseg = (ngroups - nfast + _SLOTS // 16 - 1) // (_SLOTS // 16)

        def seg_body(s, _):
            base_t = (s + 1) * (_SLOTS // 16)
            drain_flush(stage, semw)
            lax.fori_loop(
                0, jnp.minimum(ngroups - base_t, _SLOTS // 16),
                lambda t2, sb: group_body(base_t + t2, sb), 0)
            fire_flush(stage, posr, semw)
            return 0

        lax.fori_loop(0, nseg, seg_body, 0)

    def pair_body(p, carry):
        g0 = 2 * p
        chunk_dma(g0 + 1, buf1, sem1)
        drain(buf0, sem0)
        work(buf0, g0 * _W, _W, stageA, posrA, semwA)
        chunk_dma(g0 + 2, buf0, sem0)
        drain(buf1, sem1)
        work(buf1, (g0 + 1) * _W, _W, stageB, posrB, semwB)
        return carry

    lax.fori_loop(0, (_NCH - 1) // 2, pair_body, 0)
    drain(buf0, sem0)
    work(buf0, (_NCH - 1) * _W, _W, stageA, posrA, semwA)

    for a in range(8):
        pltpu.async_copy(
            poolt_hbm.at[pl.ds(8 * a, 8), pl.ds(_T0, _W)],
            buf0.at[pl.ds(8 * a, 8), :], sem0)
    drain(buf0, sem0)
    work(buf0, _RANGE, _W, stageB, posrB, semwB)
    pltpu.sync_copy(tail_hbm, tailbuf)
    work(tailbuf, _RANGE + _W, _POOL - _T1, stageA, posrA, semwA)

    drain_flush(stageA, semwA)
    drain_flush(stageB, semwB)


def kernel(ids, pool):
    poolt = pool.T
    out = _lookup(ids.astype(jnp.int32), poolt, poolt[:, _T1:])
    return out[:_BATCH, :_DIM]


# speedup vs baseline: 1.6146x; 1.6146x over previous
import functools

import jax
import jax.numpy as jnp
from jax import lax
from jax.experimental import pallas as pl
from jax.experimental.pallas import tpu as pltpu
from jax.experimental.pallas import tpu_sc as plsc

_POOL = 1000000
_DIM = 64
_BATCH = 16384

_info = plsc.get_sparse_core_info()
_NC, _NS = _info.num_cores, _info.num_subcores
_NW = _NC * _NS
_BPW = _BATCH // _NW
_CH = 128
_NCHUNK = _BPW // _CH

_mesh = plsc.VectorSubcoreMesh(core_axis_name="c", subcore_axis_name="s")


@functools.partial(
    pl.kernel,
    mesh=_mesh,
    out_type=jax.ShapeDtypeStruct((_BATCH, _DIM), jnp.float32),
    scratch_types=[
        pltpu.VMEM((_NCHUNK, _CH), jnp.int32),
        pltpu.VMEM((_BPW, _DIM), jnp.float32),
        pltpu.SemaphoreType.DMA,
    ],
    compiler_params=pltpu.CompilerParams(use_tc_tiling_on_sc=False),
)
def _lookup(ids_hbm, pool_hbm, out_hbm, idx_v, rows_v, sem):
    wid = lax.axis_index("s") * _NC + lax.axis_index("c")
    base = wid * _BPW
    pltpu.sync_copy(ids_hbm.at[wid], idx_v)
    copies = [
        pltpu.async_copy(
            pool_hbm.at[idx_v.at[j]],
            rows_v.at[pl.ds(j * _CH, _CH)],
            sem,
        )
        for j in range(_NCHUNK)
    ]
    for c in copies:
        c.wait()
    pltpu.sync_copy(rows_v, out_hbm.at[pl.ds(base, _BPW)])


def kernel(ids, pool):
    ids32 = ids.astype(jnp.int32).reshape(_NW, _NCHUNK, _CH)
    return _lookup(ids32, pool)
